# Initial kernel scaffold; baseline (speedup 1.0000x reference)
import jax
import jax.numpy as jnp
from jax.experimental import pallas as pl


def kernel(x, edge_index, W, b):
    raise NotImplementedError("write your pallas kernel here")


# speedup vs baseline: 22.1543x; 22.1543x over previous
import functools

import jax
import jax.numpy as jnp
from jax import lax
from jax.experimental import pallas as pl
from jax.experimental.pallas import tpu as pltpu
from jax.experimental.pallas import tpu_sc as plsc

N = 10000
D = 128
DH = 64
E = 320000
NPAD = 10240
NCORE = 2
NSUB = 16
RPT = NPAD // NSUB
CHUNK = 128
NCH = 160
NPADROWS = NPAD - N


def _sc_kernel_fn():
    mesh = plsc.VectorSubcoreMesh(core_axis_name="c", subcore_axis_name="s")

    @functools.partial(
        pl.kernel,
        mesh=mesh,
        compiler_params=pltpu.CompilerParams(use_tc_tiling_on_sc=False),
        out_type=(
            jax.ShapeDtypeStruct((NCORE, NPAD, DH), jnp.float32),
            jax.ShapeDtypeStruct((NCORE, NPAD, DH), jnp.float32),
            jax.ShapeDtypeStruct((NPAD,), jnp.float32),
        ),
        scratch_types=[
            pltpu.VMEM((NCH, CHUNK), jnp.int32),
            pltpu.VMEM((NCH, CHUNK), jnp.int32),
            pltpu.VMEM((CHUNK, DH), jnp.float32),
            pltpu.VMEM((64, D), jnp.float32),
            pltpu.VMEM((64, DH), jnp.float32),
            pltpu.VMEM((64, DH), jnp.float32),
            pltpu.VMEM((CHUNK,), jnp.float32),
            pltpu.VMEM((RPT,), jnp.float32),
            pltpu.VMEM_SHARED((NPAD,), jnp.float32),
            pltpu.VMEM_SHARED((NPAD, DH), jnp.float32),
            pltpu.SemaphoreType.DMA,
        ],
    )
    def sc_kernel(x_hbm, srcT_hbm, dstT_hbm,
                  s_out, xs_out, dinv_out,
                  src_v, dst_v, rows_v, xbuf_v, xh0_v, xh1_v, ones_v, dloc_v,
                  deg_sh, acc_sh, sem):
        c = lax.axis_index("c")
        s = lax.axis_index("s")
        row0 = s * RPT

        def _zero_xh0(rr, _):
            for j in range(4):
                xh0_v[rr, pl.ds(j * 16, 16)] = jnp.zeros((16,), jnp.float32)
            return 0
        lax.fori_loop(0, 64, _zero_xh0, 0)
        for j in range(8):
            ones_v[pl.ds(j * 16, 16)] = jnp.ones((16,), jnp.float32)

        def _zero_dloc(k, _):
            dloc_v[pl.ds(k * 16, 16)] = jnp.zeros((16,), jnp.float32)
            return 0
        lax.fori_loop(0, RPT // 16, _zero_dloc, 0)
        pltpu.sync_copy(dloc_v, deg_sh.at[pl.ds(row0, RPT)])

        def _zero_acc(k, _):
            pltpu.sync_copy(xh0_v, acc_sh.at[pl.ds(row0 + k * 64, 64)])
            return 0
        lax.fori_loop(0, RPT // 64, _zero_acc, 0)
        plsc.subcore_barrier()

        pltpu.sync_copy(dstT_hbm.at[pl.ds(s * NCH, NCH)], dst_v)

        def _deg(i, _):
            pltpu.sync_copy(ones_v, deg_sh.at[dst_v.at[i]], add=True)
            return 0
        lax.fori_loop(0, NCH, _deg, 0)
        plsc.subcore_barrier()

        pltpu.sync_copy(deg_sh.at[pl.ds(row0, RPT)], dloc_v)

        def _dinv(k, _):
            dv = dloc_v[pl.ds(k * 16, 16)] + 1.0
            bits = lax.bitcast_convert_type(dv, jnp.int32)
            y = lax.bitcast_convert_type(
                jnp.int32(0x5F3759DF) - (bits >> 1), jnp.float32)
            half = dv * 0.5
            y = y * (1.5 - half * y * y)
            y = y * (1.5 - half * y * y)
            y = y * (1.5 - half * y * y)
            dloc_v[pl.ds(k * 16, 16)] = y
            return 0
        lax.fori_loop(0, RPT // 16, _dinv, 0)
        pltpu.sync_copy(dloc_v, dinv_out.at[pl.ds(row0, RPT)])

        def _scale(ch, _):
            r0 = row0 + ch * 64
            pltpu.sync_copy(x_hbm.at[pl.ds(r0, 64)], xbuf_v)

            def _grp(g, _):
                dv16 = dloc_v[pl.ds(ch * 64 + g * 16, 16)]
                for rr in range(16):
                    d = dv16[rr]
                    row = g * 16 + rr
                    for j in range(4):
                        sl = pl.ds(j * 16, 16)
                        xh0_v[row, sl] = xbuf_v[row, sl] * d
                    for j in range(4):
                        sl = pl.ds(j * 16, 16)
                        xh1_v[row, sl] = xbuf_v[row, pl.ds(64 + j * 16, 16)] * d
                return 0
            lax.fori_loop(0, 4, _grp, 0)
            pltpu.sync_copy(xh0_v, xs_out.at[0, pl.ds(r0, 64)])
            pltpu.sync_copy(xh1_v, xs_out.at[1, pl.ds(r0, 64)])
            return 0
        lax.fori_loop(0, RPT // 64, _scale, 0)
        plsc.subcore_barrier()

        pltpu.sync_copy(srcT_hbm.at[pl.ds(s * NCH, NCH)], src_v)

        def _edges(i, _):
            pltpu.async_copy(xs_out.at[c].at[src_v.at[i]], rows_v, sem).wait()
            pltpu.sync_copy(rows_v, acc_sh.at[dst_v.at[i]], add=True)
            return 0
        lax.fori_loop(0, NCH, _edges, 0)
        plsc.subcore_barrier()

        def _out(k, _):
            r0 = row0 + k * CHUNK
            pltpu.sync_copy(acc_sh.at[pl.ds(r0, CHUNK)], rows_v)
            pltpu.sync_copy(rows_v, s_out.at[c, pl.ds(r0, CHUNK)])
            return 0
        lax.fori_loop(0, RPT // CHUNK, _out, 0)

    return sc_kernel


_SC_KERNEL = _sc_kernel_fn()

_TC_ROWS = 1280


def _tc_body(s_ref, xs_ref, dinv_ref, w_ref, b_ref, o_ref):
    u = jnp.concatenate(
        [s_ref[0] + xs_ref[0], s_ref[1] + xs_ref[1]], axis=1)
    acc = jnp.dot(u, w_ref[...], preferred_element_type=jnp.float32)
    o_ref[...] = acc * dinv_ref[...] + b_ref[...]


def kernel(x, edge_index, W, b):
    src = edge_index[0].astype(jnp.int32)
    dst = edge_index[1].astype(jnp.int32)
    ept_real = E // NSUB
    npad_e = NCH * CHUNK - ept_real
    pad = N + jnp.arange(npad_e, dtype=jnp.int32) % NPADROWS
    pad = jnp.broadcast_to(pad, (NSUB, npad_e))
    srcT = jnp.concatenate([src.reshape(NSUB, ept_real), pad], axis=1)
    dstT = jnp.concatenate([dst.reshape(NSUB, ept_real), pad], axis=1)
    srcT = srcT.reshape(NSUB * NCH, CHUNK)
    dstT = dstT.reshape(NSUB * NCH, CHUNK)
    x_pad = jnp.concatenate(
        [x, jnp.zeros((NPAD - N, D), jnp.float32)], axis=0)

    S, xs, dinv = _SC_KERNEL(x_pad, srcT, dstT)

    out_pad = pl.pallas_call(
        _tc_body,
        grid=(NPAD // _TC_ROWS,),
        in_specs=[
            pl.BlockSpec((NCORE, _TC_ROWS, DH), lambda i: (0, i, 0)),
            pl.BlockSpec((NCORE, _TC_ROWS, DH), lambda i: (0, i, 0)),
            pl.BlockSpec((_TC_ROWS, 1), lambda i: (i, 0)),
            pl.BlockSpec((D, D), lambda i: (0, 0)),
            pl.BlockSpec((1, D), lambda i: (0, 0)),
        ],
        out_specs=pl.BlockSpec((_TC_ROWS, D), lambda i: (i, 0)),
        out_shape=jax.ShapeDtypeStruct((NPAD, D), jnp.float32),
    )(S, xs, dinv.reshape(NPAD, 1), W, b.reshape(1, D))

    return out_pad[:N]


# speedup vs baseline: 28.3269x; 1.2786x over previous
import functools

import jax
import jax.numpy as jnp
from jax import lax
from jax.experimental import pallas as pl
from jax.experimental.pallas import tpu as pltpu
from jax.experimental.pallas import tpu_sc as plsc

N = 10000
D = 128
DH = 64
E = 320000
NPAD = 10240
NCORE = 2
NSUB = 16
RPT = NPAD // NSUB
CHUNK = 128
NCH = 160
NPADROWS = NPAD - N
NBUF = 2


def _sc_kernel_fn():
    mesh = plsc.VectorSubcoreMesh(core_axis_name="c", subcore_axis_name="s")

    @functools.partial(
        pl.kernel,
        mesh=mesh,
        compiler_params=pltpu.CompilerParams(use_tc_tiling_on_sc=False),
        out_type=(
            jax.ShapeDtypeStruct((NCORE, NPAD, DH), jnp.float32),
            jax.ShapeDtypeStruct((NCORE, NPAD, DH), jnp.float32),
            jax.ShapeDtypeStruct((NPAD,), jnp.float32),
        ),
        scratch_types=[
            pltpu.VMEM((NCH, CHUNK), jnp.int32),
            pltpu.VMEM((NCH, CHUNK), jnp.int32),
            pltpu.VMEM((CHUNK, DH), jnp.float32),
            pltpu.VMEM((CHUNK, DH), jnp.float32),
            pltpu.VMEM((64, D), jnp.float32),
            pltpu.VMEM((64, DH), jnp.float32),
            pltpu.VMEM((64, DH), jnp.float32),
            pltpu.VMEM((CHUNK,), jnp.float32),
            pltpu.VMEM((RPT,), jnp.float32),
            pltpu.VMEM_SHARED((NPAD,), jnp.float32),
            pltpu.VMEM_SHARED((NPAD, DH), jnp.float32),
            pltpu.SemaphoreType.DMA,
            pltpu.SemaphoreType.DMA,
            pltpu.SemaphoreType.DMA,
            pltpu.SemaphoreType.DMA,
            pltpu.SemaphoreType.DMA,
        ],
    )
    def sc_kernel(x_hbm, srcT_hbm, dstT_hbm,
                  s_out, xs_out, dinv_out,
                  src_v, dst_v, rb0, rb1, xbuf_v, xh0_v, xh1_v,
                  ones_v, dloc_v, deg_sh, acc_sh,
                  dsem, gs0, gs1, ss0, ss1):
        c = lax.axis_index("c")
        s = lax.axis_index("s")
        row0 = s * RPT
        rbufs = [rb0, rb1]
        gsems = [gs0, gs1]
        ssems = [ss0, ss1]

        for j in range(8):
            ones_v[pl.ds(j * 16, 16)] = jnp.ones((16,), jnp.float32)

        def _zero_dloc(k, _):
            dloc_v[pl.ds(k * 16, 16)] = jnp.zeros((16,), jnp.float32)
            return 0
        lax.fori_loop(0, RPT // 16, _zero_dloc, 0)
        pltpu.sync_copy(dloc_v, deg_sh.at[pl.ds(row0, RPT)])
        plsc.subcore_barrier()

        pltpu.sync_copy(dstT_hbm.at[pl.ds(s * NCH, NCH)], dst_v)

        def _deg(i, _):
            pltpu.async_copy(ones_v, deg_sh.at[dst_v.at[i]], dsem, add=True)

            @pl.when(i >= 8)
            def _():
                pltpu.make_async_copy(
                    ones_v, deg_sh.at[dst_v.at[i - 8]], dsem).wait()
            return 0
        lax.fori_loop(0, NCH, _deg, 0)
        for i in range(NCH - 8, NCH):
            pltpu.make_async_copy(
                ones_v, deg_sh.at[dst_v.at[i]], dsem).wait()
        plsc.subcore_barrier()

        pltpu.sync_copy(deg_sh.at[pl.ds(row0, RPT)], dloc_v)

        def _dinv(k, _):
            dv = dloc_v[pl.ds(k * 16, 16)] + 1.0
            bits = lax.bitcast_convert_type(dv, jnp.int32)
            y = lax.bitcast_convert_type(
                jnp.int32(0x5F3759DF) - (bits >> 1), jnp.float32)
            half = dv * 0.5
            y = y * (1.5 - half * y * y)
            y = y * (1.5 - half * y * y)
            y = y * (1.5 - half * y * y)
            dloc_v[pl.ds(k * 16, 16)] = y
            return 0
        lax.fori_loop(0, RPT // 16, _dinv, 0)
        pltpu.sync_copy(dloc_v, dinv_out.at[pl.ds(row0, RPT)])

        def _scale(ch, _):
            r0 = row0 + ch * 64
            pltpu.sync_copy(x_hbm.at[pl.ds(r0, 64)], xbuf_v)

            def _grp(g, _):
                dv16 = dloc_v[pl.ds(ch * 64 + g * 16, 16)]
                for rr in range(16):
                    d = dv16[rr]
                    row = g * 16 + rr
                    for j in range(4):
                        sl = pl.ds(j * 16, 16)
                        xh0_v[row, sl] = xbuf_v[row, sl] * d
                    for j in range(4):
                        sl = pl.ds(j * 16, 16)
                        xh1_v[row, sl] = xbuf_v[row, pl.ds(64 + j * 16, 16)] * d
                return 0
            lax.fori_loop(0, 4, _grp, 0)
            pltpu.sync_copy(xh0_v, xs_out.at[0, pl.ds(r0, 64)])
            pltpu.sync_copy(xh1_v, xs_out.at[1, pl.ds(r0, 64)])

            @pl.when(c == 0)
            def _():
                pltpu.sync_copy(xh0_v, acc_sh.at[pl.ds(r0, 64)])

            @pl.when(c == 1)
            def _():
                pltpu.sync_copy(xh1_v, acc_sh.at[pl.ds(r0, 64)])
            return 0
        lax.fori_loop(0, RPT // 64, _scale, 0)
        plsc.subcore_barrier()

        pltpu.sync_copy(srcT_hbm.at[pl.ds(s * NCH, NCH)], src_v)

        def _gstart(k, i):
            pltpu.async_copy(
                xs_out.at[c].at[src_v.at[i]], rbufs[k], gsems[k])

        def _gwait(k, i):
            pltpu.make_async_copy(
                xs_out.at[c].at[src_v.at[i]], rbufs[k], gsems[k]).wait()

        def _sstart(k, i):
            pltpu.async_copy(
                rbufs[k], acc_sh.at[dst_v.at[i]], ssems[k], add=True)

        def _swait(k, i):
            pltpu.make_async_copy(
                rbufs[k], acc_sh.at[dst_v.at[i]], ssems[k]).wait()

        for k in range(NBUF):
            _gstart(k, k)

        def _quad(p, _):
            i0 = NBUF * p
            for k in range(NBUF):
                i = i0 + k
                _gwait(k, i)
                _sstart(k, i)
                km1 = (k - 1) % NBUF
                if km1 == NBUF - 1:
                    @pl.when(p > 0)
                    def _():
                        _swait(km1, i - 1)
                        _gstart(km1, i + NBUF - 1)
                else:
                    @pl.when(i + NBUF - 1 < NCH)
                    def _():
                        _swait(km1, i - 1)
                        _gstart(km1, i + NBUF - 1)
            return 0
        lax.fori_loop(0, NCH // NBUF, _quad, 0)
        for k in range(NBUF - 1):
            _swait(k, NCH - NBUF + k)
        _swait(NBUF - 1, NCH - 1)
        plsc.subcore_barrier()

        def _out(k, _):
            r0 = row0 + k * CHUNK
            pltpu.sync_copy(acc_sh.at[pl.ds(r0, CHUNK)], rb0)
            pltpu.sync_copy(rb0, s_out.at[c, pl.ds(r0, CHUNK)])
            return 0
        lax.fori_loop(0, RPT // CHUNK, _out, 0)

    return sc_kernel


_SC_KERNEL = _sc_kernel_fn()

_TC_ROWS = 2000


def _tc_body(s_ref, dinv_ref, w_ref, b_ref, o_ref):
    u = jnp.concatenate([s_ref[0], s_ref[1]], axis=1)
    acc = jnp.dot(u, w_ref[...], preferred_element_type=jnp.float32)
    o_ref[...] = acc * dinv_ref[...] + b_ref[...]


def kernel(x, edge_index, W, b):
    src = edge_index[0].astype(jnp.int32)
    dst = edge_index[1].astype(jnp.int32)
    ept_real = E // NSUB
    npad_e = NCH * CHUNK - ept_real
    pad = N + jnp.arange(npad_e, dtype=jnp.int32) % NPADROWS
    pad = jnp.broadcast_to(pad, (NSUB, npad_e))
    srcT = jnp.concatenate([src.reshape(NSUB, ept_real), pad], axis=1)
    dstT = jnp.concatenate([dst.reshape(NSUB, ept_real), pad], axis=1)
    srcT = srcT.reshape(NSUB * NCH, CHUNK)
    dstT = dstT.reshape(NSUB * NCH, CHUNK)
    x_pad = jnp.concatenate(
        [x, jnp.zeros((NPAD - N, D), jnp.float32)], axis=0)

    S, _, dinv = _SC_KERNEL(x_pad, srcT, dstT)

    out = pl.pallas_call(
        _tc_body,
        grid=(N // _TC_ROWS,),
        in_specs=[
            pl.BlockSpec((NCORE, _TC_ROWS, DH), lambda i: (0, i, 0)),
            pl.BlockSpec((_TC_ROWS, 1), lambda i: (i, 0)),
            pl.BlockSpec((D, D), lambda i: (0, 0)),
            pl.BlockSpec((1, D), lambda i: (0, 0)),
        ],
        out_specs=pl.BlockSpec((_TC_ROWS, D), lambda i: (i, 0)),
        out_shape=jax.ShapeDtypeStruct((N, D), jnp.float32),
    )(S, dinv.reshape(NPAD, 1)[:N], W, b.reshape(1, D))

    return out
